# Initial kernel scaffold; baseline (speedup 1.0000x reference)
import jax
import jax.numpy as jnp
from jax.experimental import pallas as pl


def kernel(input_ids, token_type_ids, word_emb, pos_emb, type_emb, ln_gamma, ln_beta):
    raise NotImplementedError("write your pallas kernel here")


# speedup vs baseline: 1.0202x; 1.0202x over previous
import functools

import jax
import jax.numpy as jnp
from jax import lax
from jax.experimental import pallas as pl
from jax.experimental.pallas import tpu as pltpu
from jax.experimental.pallas import tpu_sc as plsc

D = 768
LANES = 16
DV = D // LANES
NC, NS = 2, 16
NW = NC * NS
CHUNK = 64
EPS = 1e-12


def _ptsum_body(pos_ref, type_ref, out_ref):
    out_ref[...] = pos_ref[...][:, None, :] + type_ref[...][None, :, :]


def _make_sc_kernel(n_tok):
    tpw = n_tok // NW
    nch = tpw // CHUNK
    mesh = plsc.VectorSubcoreMesh(
        core_axis_name="c", subcore_axis_name="s",
        num_cores=NC, num_subcores=NS)

    @functools.partial(
        pl.kernel,
        out_type=jax.ShapeDtypeStruct((n_tok, D), jnp.float32),
        mesh=mesh,
        compiler_params=pltpu.CompilerParams(needs_layout_passes=False),
        scratch_types=[
            pltpu.VMEM((CHUNK,), jnp.int32),
            pltpu.VMEM((CHUNK,), jnp.int32),
            pltpu.VMEM((CHUNK, D), jnp.float32),
            pltpu.VMEM((CHUNK, D), jnp.float32),
            pltpu.VMEM((D,), jnp.float32),
            pltpu.VMEM((D,), jnp.float32),
            pltpu.SemaphoreType.DMA,
            pltpu.SemaphoreType.DMA,
        ],
    )
    def sc_kernel(ids_hbm, gidx_hbm, wtab_hbm, ptab_hbm, gam_hbm, bet_hbm,
                  out_hbm, idx_v, gidx_v, wbuf, pbuf, gam_v, bet_v, sem1, sem2):
        wid = lax.axis_index("s") * NC + lax.axis_index("c")
        pltpu.sync_copy(gam_hbm, gam_v)
        pltpu.sync_copy(bet_hbm, bet_v)
        base0 = wid * tpw

        def chunk_body(ci, carry):
            base = base0 + ci * CHUNK
            pltpu.sync_copy(ids_hbm.at[pl.ds(base, CHUNK)], idx_v)
            pltpu.sync_copy(gidx_hbm.at[pl.ds(base, CHUNK)], gidx_v)
            cp1 = pltpu.async_copy(wtab_hbm.at[idx_v], wbuf, sem1)
            cp2 = pltpu.async_copy(ptab_hbm.at[gidx_v], pbuf, sem2)
            cp1.wait()
            cp2.wait()

            def tok_body(t, carry2):
                acc = jnp.zeros((LANES,), jnp.float32)
                acc2 = jnp.zeros((LANES,), jnp.float32)
                for d in range(DV):
                    sl = pl.ds(d * LANES, LANES)
                    e = wbuf[t, sl] + pbuf[t, sl]
                    wbuf[t, sl] = e
                    acc = acc + e
                    acc2 = acc2 + e * e
                s1 = jnp.broadcast_to(jnp.sum(acc), (LANES,))
                s2 = jnp.broadcast_to(jnp.sum(acc2), (LANES,))
                meanv = s1 * (1.0 / D)
                varv = s2 * (1.0 / D) - meanv * meanv
                x = varv + EPS
                bits = plsc.bitcast(x, jnp.int32)
                bits = jnp.int32(0x5F3759DF) - lax.shift_right_logical(
                    bits, jnp.full((LANES,), 1, jnp.int32))
                y = plsc.bitcast(bits, jnp.float32)
                for _ in range(4):
                    y = y * (1.5 - 0.5 * x * y * y)
                for d in range(DV):
                    sl = pl.ds(d * LANES, LANES)
                    e = wbuf[t, sl]
                    wbuf[t, sl] = (e - meanv) * y * gam_v[sl] + bet_v[sl]
                return carry2

            lax.fori_loop(0, CHUNK, tok_body, 0)
            pltpu.sync_copy(wbuf, out_hbm.at[pl.ds(base, CHUNK)])
            return carry

        lax.fori_loop(0, nch, chunk_body, 0)

    return sc_kernel


def kernel(input_ids, token_type_ids, word_emb, pos_emb, type_emb,
           ln_gamma, ln_beta):
    B, S = input_ids.shape
    n_tok = B * S
    n_types = type_emb.shape[0]
    ids = input_ids.reshape(-1).astype(jnp.int32)
    gidx = (n_types * jnp.arange(S, dtype=jnp.int32)[None, :]
            + token_type_ids.astype(jnp.int32)).reshape(-1)
    ptab = pl.pallas_call(
        _ptsum_body,
        out_shape=jax.ShapeDtypeStruct(
            (pos_emb.shape[0], n_types, D), jnp.float32),
    )(pos_emb, type_emb).reshape(-1, D)
    out = _make_sc_kernel(n_tok)(ids, gidx, word_emb, ptab, ln_gamma, ln_beta)
    return out.reshape(B, S, D)
